# Initial kernel scaffold; baseline (speedup 1.0000x reference)
import jax
import jax.numpy as jnp
from jax.experimental import pallas as pl


def kernel(x, edge_index, W1, b1, W2, b2, W_ih, W_hh, b_ih, b_hh, W_fc, b_fc):
    raise NotImplementedError("write your pallas kernel here")


# speedup vs baseline: 23.1817x; 23.1817x over previous
import jax
import jax.numpy as jnp
from jax import lax
from jax.experimental import pallas as pl
from jax.experimental.pallas import tpu as pltpu
from jax.experimental.pallas import tpu_sc as plsc

N = 10000
E = 320000
D = 128
H = 64
G4 = 4 * H

NT = 16
NP = 10240
CH = E // NT
BE = 2000
NBK = CH // BE
GB = BE // 16
NPC = NP // NT
GN = NPC // 16
SB = 128
SPT = SB // NT
BIG = 0x3FFFFFFF

f32 = jnp.float32
i32 = jnp.int32


def _rsqrt_nr(x):
    ib = plsc.bitcast(x, i32)
    y = plsc.bitcast(jnp.int32(0x5F3759DF) - (ib >> 1), f32)
    for _ in range(3):
        y = y * (1.5 - 0.5 * x * y * y)
    return y


def _dx(v, i):
    lane = lax.iota(i32, 16) == i
    return jnp.sum(jnp.where(lane, v, jnp.zeros_like(v)))


def _sc_body(xt, srcs, dsts, w1, b1, w2, b2, wihT, bih, bhh, wfc, bfc,
             out,
             es_blk, ed_blk, deg_l, cnt_l, slot_l, tacc,
             cstage, sstage, vstage, tstage, cbufs,
             w1_l, b1_l, rows, accv, xrow, tvec, u_l,
             wblk, w2blk, b2_l, bih_l, bhh_l, wfc_l, bfc_l, zbuf, gbuf,
             ustage, ybuf,
             hstage, astage, acc_m, dis_m, slot_m, cshare, u_sh,
             sem):
    wid = lax.axis_index("s")
    ebase = wid * CH
    base = wid * NPC
    rid = lax.iota(i32, 16)
    zv = jnp.zeros((16,), f32)

    pltpu.sync_copy(w1, w1_l)
    pltpu.sync_copy(b1, b1_l)

    @pl.when(wid == 0)
    def _():
        pltpu.sync_copy(b2, b2_l)
        pltpu.sync_copy(bih, bih_l)
        pltpu.sync_copy(bhh, bhh_l)
        pltpu.sync_copy(wfc, wfc_l)
        pltpu.sync_copy(bfc, bfc_l)

    for q in range(H // 16):
        u_l[pl.ds(q * 16, 16)] = zv

    def _zhr(r, _):
        def _zh(j, _):
            deg_l[r, pl.ds(j * 16, 16)] = zv
            cnt_l[r, pl.ds(j * 16, 16)] = zv
            return 0
        lax.fori_loop(0, GN, _zh, 0)
        return 0
    lax.fori_loop(0, NT, _zhr, 0)

    def _blka(blk, _):
        pltpu.sync_copy(srcs.at[pl.ds(ebase + blk * BE, BE)], es_blk)
        pltpu.sync_copy(dsts.at[pl.ds(ebase + blk * BE, BE)], ed_blk)

        def _pha(g, _):
            dvec = ed_blk[pl.ds(g * 16, 16)]
            svec = es_blk[pl.ds(g * 16, 16)]
            drow = dvec // NPC
            dcol = dvec - drow * NPC
            c, last = plsc.scan_count(dvec)
            plsc.addupdate_scatter(deg_l, [drow, dcol], c.astype(f32),
                                   mask=last)
            m0 = dvec == 0
            srow = svec // NPC
            scol = svec - srow * NPC
            c2, last2 = plsc.scan_count(svec, mask=m0)
            plsc.addupdate_scatter(cnt_l, [srow, scol], c2.astype(f32),
                                   mask=jnp.logical_and(last2, m0))
            return 0
        lax.fori_loop(0, GB, _pha, 0)
        return 0
    lax.fori_loop(0, NBK, _blka, 0)

    pltpu.sync_copy(deg_l, hstage.at[wid])
    plsc.subcore_barrier()

    def _zvst(j, _):
        vstage[pl.ds(j * 16, 16)] = zv
        return 0
    lax.fori_loop(0, GN, _zvst, 0)

    def _mk1(k, _):
        pltpu.sync_copy(hstage.at[k, wid], tstage)

        def _acc1(j, _):
            vstage[pl.ds(j * 16, 16)] = (
                vstage[pl.ds(j * 16, 16)] + tstage[pl.ds(j * 16, 16)])
            return 0
        lax.fori_loop(0, GN, _acc1, 0)
        return 0
    lax.fori_loop(0, NT, _mk1, 0)

    def _mdis(j, _):
        vstage[pl.ds(j * 16, 16)] = _rsqrt_nr(vstage[pl.ds(j * 16, 16)] + 1.0)
        return 0
    lax.fori_loop(0, GN, _mdis, 0)
    pltpu.sync_copy(vstage, dis_m.at[wid])
    plsc.subcore_barrier()

    pltpu.sync_copy(cnt_l, hstage.at[wid])
    plsc.subcore_barrier()

    def _zcst(j, _):
        cstage[pl.ds(j * 16, 16)] = zv
        return 0
    lax.fori_loop(0, GN, _zcst, 0)

    def _mk2(k, _):
        pltpu.sync_copy(hstage.at[k, wid], tstage)

        def _acc2(j, _):
            cstage[pl.ds(j * 16, 16)] = (
                cstage[pl.ds(j * 16, 16)] + tstage[pl.ds(j * 16, 16)])
            return 0
        lax.fori_loop(0, GN, _acc2, 0)
        return 0
    lax.fori_loop(0, NT, _mk2, 0)

    @pl.when(wid == 0)
    def _():
        c16 = cstage[pl.ds(0, 16)]
        cstage[pl.ds(0, 16)] = c16 + jnp.where(rid == 0, 1.0, 0.0)

    def _cflag(j, t):
        c16 = cstage[pl.ds(j * 16, 16)]
        cum = plsc.cumsum(jnp.where(c16 > 0.0, 1, 0))
        return t + cum[15]
    mycnt = lax.fori_loop(0, GN, _cflag, jnp.int32(0))
    sstage[pl.ds(0, 16)] = jnp.where(rid == 0, mycnt, 0)
    pltpu.sync_copy(sstage.at[pl.ds(0, 16)], cshare.at[wid])
    plsc.subcore_barrier()

    pltpu.sync_copy(cshare, cbufs)

    def _pfx(k, to):
        t, o = to
        ck = cbufs[k, pl.ds(0, 16)][0]
        return (t + ck, o + jnp.where(k < wid, ck, 0))
    total, offset = lax.fori_loop(0, NT, _pfx, (jnp.int32(0), jnp.int32(0)))

    def _sloop(j, run):
        c16 = cstage[pl.ds(j * 16, 16)]
        flag = c16 > 0.0
        cum = plsc.cumsum(jnp.where(flag, 1, 0))
        sstage[pl.ds(j * 16, 16)] = jnp.where(flag, run + cum - 1, BIG)
        return run + cum[15]
    lax.fori_loop(0, GN, _sloop, offset)
    pltpu.sync_copy(sstage, slot_m.at[wid])
    plsc.subcore_barrier()

    pltpu.sync_copy(dis_m, deg_l)
    pltpu.sync_copy(slot_m, slot_l)

    nbatch = (total + (SB - 1)) // SB

    def _batch(b, _):
        lo = b * SB

        def _zt(r, _):
            for q in range(D // 16):
                tacc[r, pl.ds(q * 16, 16)] = zv
            return 0
        lax.fori_loop(0, SB, _zt, 0)

        def _blkc(blk, _):
            pltpu.sync_copy(srcs.at[pl.ds(ebase + blk * BE, BE)], es_blk)
            pltpu.sync_copy(dsts.at[pl.ds(ebase + blk * BE, BE)], ed_blk)

            def _phc(g, _):
                dvec = ed_blk[pl.ds(g * 16, 16)]
                drow = dvec // NPC
                dcol = dvec - drow * NPC
                sl = plsc.load_gather(slot_l, [drow, dcol])
                inb = jnp.logical_and(sl >= lo, sl < lo + SB)
                anyf = jnp.max(jnp.where(inb, 1, 0))

                @pl.when(anyf > 0)
                def _():
                    svec = es_blk[pl.ds(g * 16, 16)]
                    srow = svec // NPC
                    scol = svec - srow * NPC
                    wv = plsc.load_gather(deg_l, [srow, scol])
                    wv = jnp.where(inb, wv, 0.0)
                    sidx = jnp.where(inb, sl - lo, 0)
                    pltpu.async_copy(xt.at[svec], rows, sem).wait()

                    def _lane(i, _):
                        wi = _dx(wv, i)
                        ti = _dx(sidx, i)
                        for q in range(D // 16):
                            tacc[ti, pl.ds(q * 16, 16)] = (
                                tacc[ti, pl.ds(q * 16, 16)]
                                + wi * rows[i, pl.ds(q * 16, 16)])
                        return 0
                    lax.fori_loop(0, 16, _lane, 0)
                return 0
            lax.fori_loop(0, GB, _phc, 0)
            return 0
        lax.fori_loop(0, NBK, _blkc, 0)

        pltpu.sync_copy(tacc, astage.at[wid])
        plsc.subcore_barrier()

        def _mrow(r, _):
            s = wid * SPT + r
            for q in range(D // 16):
                accv[pl.ds(q * 16, 16)] = zv

            def _mk(k, _):
                pltpu.sync_copy(astage.at[k, s], xrow)
                for q in range(D // 16):
                    accv[pl.ds(q * 16, 16)] = (
                        accv[pl.ds(q * 16, 16)] + xrow[pl.ds(q * 16, 16)])
                return 0
            lax.fori_loop(0, NT, _mk, 0)
            pltpu.sync_copy(accv, acc_m.at[s])
            return 0
        lax.fori_loop(0, SPT, _mrow, 0)
        plsc.subcore_barrier()

        def _phd(j, _):
            sl16 = slot_l[wid, pl.ds(j * 16, 16)]
            inb = jnp.logical_and(sl16 >= lo, sl16 < lo + SB)
            inb32 = jnp.where(inb, 1, 0)
            anyf = jnp.max(inb32)

            @pl.when(anyf > 0)
            def _():
                cvec = cstage[pl.ds(j * 16, 16)]
                dvec16 = deg_l[wid, pl.ds(j * 16, 16)]

                def _lane(i, _):
                    @pl.when(_dx(inb32, i) > 0)
                    def _():
                        v = base + j * 16 + i
                        pltpu.sync_copy(acc_m.at[_dx(sl16, i) - lo], accv)
                        pltpu.sync_copy(xt.at[v], xrow)
                        disv = _dx(dvec16, i)
                        for q in range(D // 16):
                            accv[pl.ds(q * 16, 16)] = (
                                accv[pl.ds(q * 16, 16)]
                                + disv * xrow[pl.ds(q * 16, 16)])
                        for q in range(H // 16):
                            tvec[pl.ds(q * 16, 16)] = zv

                        def _kl(kg, _):
                            a16 = accv[pl.ds(kg * 16, 16)]
                            for il in range(16):
                                ak = a16[il]
                                for q in range(H // 16):
                                    tvec[pl.ds(q * 16, 16)] = (
                                        tvec[pl.ds(q * 16, 16)]
                                        + ak * w1_l[kg * 16 + il,
                                                    pl.ds(q * 16, 16)])
                            return 0
                        lax.fori_loop(0, D // 16, _kl, 0)
                        sc = _dx(cvec, i) * disv
                        for q in range(H // 16):
                            hq = jnp.maximum(
                                disv * tvec[pl.ds(q * 16, 16)]
                                + b1_l[pl.ds(q * 16, 16)], 0.0)
                            u_l[pl.ds(q * 16, 16)] = (
                                u_l[pl.ds(q * 16, 16)] + sc * hq)
                    return 0
                lax.fori_loop(0, 16, _lane, 0)
            return 0
        lax.fori_loop(0, GN, _phd, 0)
        plsc.subcore_barrier()
        return 0
    lax.fori_loop(0, nbatch, _batch, 0)

    pltpu.sync_copy(u_l, u_sh.at[wid])
    plsc.subcore_barrier()

    @pl.when(wid == 0)
    def _():
        pltpu.sync_copy(u_sh, ustage)
        for q in range(H // 16):
            s = jnp.zeros((16,), f32)
            for k in range(NT):
                s = s + ustage[k, pl.ds(q * 16, 16)]
            u_l[pl.ds(q * 16, 16)] = s
        dis0 = deg_l[0, pl.ds(0, 16)][0]

        for q in range(H // 16):
            zbuf[pl.ds(q * 16, 16)] = zv

        def _k2(kg, _):
            pltpu.sync_copy(w2.at[pl.ds(kg * 16, 16)], w2blk)
            u16 = u_l[pl.ds(kg * 16, 16)]

            def _il2(il, _):
                uk = _dx(u16, il)
                for q in range(H // 16):
                    zbuf[pl.ds(q * 16, 16)] = (
                        zbuf[pl.ds(q * 16, 16)]
                        + uk * w2blk[il, pl.ds(q * 16, 16)])
                return 0
            lax.fori_loop(0, 16, _il2, 0)
            return 0
        lax.fori_loop(0, H // 16, _k2, 0)
        for q in range(H // 16):
            zbuf[pl.ds(q * 16, 16)] = (
                dis0 * zbuf[pl.ds(q * 16, 16)] + b2_l[pl.ds(q * 16, 16)])

        for r in range(G4 // 16):
            gbuf[pl.ds(r * 16, 16)] = (
                bih_l[pl.ds(r * 16, 16)] + bhh_l[pl.ds(r * 16, 16)])

        def _k3(kg, _):
            pltpu.sync_copy(wihT.at[pl.ds(kg * 16, 16)], wblk)
            z16 = zbuf[pl.ds(kg * 16, 16)]

            def _il3(il, _):
                zk = _dx(z16, il)
                for r in range(G4 // 16):
                    gbuf[pl.ds(r * 16, 16)] = (
                        gbuf[pl.ds(r * 16, 16)]
                        + zk * wblk[il, pl.ds(r * 16, 16)])
                return 0
            lax.fori_loop(0, 16, _il3, 0)
            return 0
        lax.fori_loop(0, H // 16, _k3, 0)

        def _sig(t):
            return 1.0 / (1.0 + jnp.exp(-t))

        def _tnh(t):
            return 1.0 - 2.0 / (jnp.exp(2.0 * t) + 1.0)

        yacc = jnp.zeros((16,), f32)
        for q in range(H // 16):
            ig = _sig(gbuf[pl.ds(q * 16, 16)])
            gg = _tnh(gbuf[pl.ds(2 * H + q * 16, 16)])
            og = _sig(gbuf[pl.ds(3 * H + q * 16, 16)])
            hv = og * _tnh(ig * gg)
            yacc = yacc + hv * wfc_l[pl.ds(q * 16, 16)]
        y = jnp.sum(yacc) + bfc_l[...][0]
        ybuf[...] = jnp.where(rid == 0, y, 0.0)
        pltpu.sync_copy(ybuf, out)


_sc_kernel = pl.kernel(
    _sc_body,
    out_type=jax.ShapeDtypeStruct((16,), f32),
    mesh=plsc.VectorSubcoreMesh(
        core_axis_name="c", subcore_axis_name="s", num_cores=1),
    scratch_types=[
        pltpu.VMEM((BE,), i32),
        pltpu.VMEM((BE,), i32),
        pltpu.VMEM((NT, NPC), f32),
        pltpu.VMEM((NT, NPC), f32),
        pltpu.VMEM((NT, NPC), i32),
        pltpu.VMEM((SB, D), f32),
        pltpu.VMEM((NPC,), f32),
        pltpu.VMEM((NPC,), i32),
        pltpu.VMEM((NPC,), f32),
        pltpu.VMEM((NPC,), f32),
        pltpu.VMEM((NT, 16), i32),
        pltpu.VMEM((D, H), f32),
        pltpu.VMEM((H,), f32),
        pltpu.VMEM((16, D), f32),
        pltpu.VMEM((D,), f32),
        pltpu.VMEM((D,), f32),
        pltpu.VMEM((H,), f32),
        pltpu.VMEM((H,), f32),
        pltpu.VMEM((16, G4), f32),
        pltpu.VMEM((16, H), f32),
        pltpu.VMEM((H,), f32),
        pltpu.VMEM((G4,), f32),
        pltpu.VMEM((G4,), f32),
        pltpu.VMEM((H,), f32),
        pltpu.VMEM((16,), f32),
        pltpu.VMEM((H,), f32),
        pltpu.VMEM((G4,), f32),
        pltpu.VMEM((NT, H), f32),
        pltpu.VMEM((16,), f32),
        pltpu.VMEM_SHARED((NT, NT, NPC), f32),
        pltpu.VMEM_SHARED((NT, SB, D), f32),
        pltpu.VMEM_SHARED((SB, D), f32),
        pltpu.VMEM_SHARED((NT, NPC), f32),
        pltpu.VMEM_SHARED((NT, NPC), i32),
        pltpu.VMEM_SHARED((NT, 16), i32),
        pltpu.VMEM_SHARED((NT, H), f32),
        pltpu.SemaphoreType.DMA,
    ],
    compiler_params=pltpu.CompilerParams(needs_layout_passes=False),
)


def kernel(x, edge_index, W1, b1, W2, b2, W_ih, W_hh, b_ih, b_hh, W_fc, b_fc):
    del W_hh
    xt = jnp.asarray(x.T)
    src = edge_index[0]
    dst = edge_index[1]
    wihT = jnp.asarray(W_ih.T)
    wfc = W_fc.reshape(H)
    bfc16 = jnp.broadcast_to(b_fc, (16,))
    res = _sc_kernel(xt, src, dst, W1, b1, W2, b2, wihT, b_ih, b_hh,
                     wfc, bfc16)
    return res[0:1].reshape(1, 1, 1)
